# Initial kernel scaffold; baseline (speedup 1.0000x reference)
import jax
import jax.numpy as jnp
from jax.experimental import pallas as pl


def kernel(x, edge_index, W, b):
    raise NotImplementedError("write your pallas kernel here")


# speedup vs baseline: 11.9007x; 11.9007x over previous
import functools

import jax
import jax.numpy as jnp
from jax import lax
from jax.experimental import pallas as pl
from jax.experimental.pallas import tpu as pltpu
from jax.experimental.pallas import tpu_sc as plsc

N = 10000
E = 320000
D = 128
NC = 2
NS = 16
NW = NC * NS
NP = 10240
RPT = NP // NS
EP = E // NW
C = 80
NCH = EP // C
NB = 16
RB = NP // NB

_sc_mesh = plsc.VectorSubcoreMesh(
    core_axis_name="c", subcore_axis_name="s", num_cores=NC, num_subcores=NS
)


def _deg_body(col_hbm, deg0_hbm, deg1_hbm, acc, idx_v, ones_v, zero_v):
    c = lax.axis_index("c")
    s = lax.axis_index("s")
    wid = c * NS + s

    @pl.loop(0, RPT // 16)
    def _zfill(i):
        zero_v[pl.ds(i * 16, 16)] = jnp.zeros((16,), jnp.float32)

    @pl.loop(0, C // 16)
    def _ofill(i):
        ones_v[pl.ds(i * 16, 16)] = jnp.ones((16,), jnp.float32)

    pltpu.sync_copy(zero_v, acc.at[pl.ds(s * RPT, RPT)])
    plsc.subcore_barrier()

    base = wid * EP

    @pl.loop(0, NCH)
    def _chunk(k):
        pltpu.sync_copy(col_hbm.at[pl.ds(base + k * C, C)], idx_v)
        pltpu.sync_copy(ones_v, acc.at[idx_v], add=True)

    plsc.subcore_barrier()

    @pl.when(c == 0)
    def _dump0():
        pltpu.sync_copy(acc.at[pl.ds(s * RPT, RPT)], deg0_hbm.at[pl.ds(s * RPT, RPT)])

    @pl.when(c == 1)
    def _dump1():
        pltpu.sync_copy(acc.at[pl.ds(s * RPT, RPT)], deg1_hbm.at[pl.ds(s * RPT, RPT)])


_deg_kernel = pl.kernel(
    _deg_body,
    out_type=[
        jax.ShapeDtypeStruct((NP,), jnp.float32),
        jax.ShapeDtypeStruct((NP,), jnp.float32),
    ],
    mesh=_sc_mesh,
    scratch_types=[
        pltpu.VMEM_SHARED((NP,), jnp.float32),
        pltpu.VMEM((C,), jnp.int32),
        pltpu.VMEM((C,), jnp.float32),
        pltpu.VMEM((RPT,), jnp.float32),
    ],
)


def _spmm_body(hs_hbm, row_hbm, col_hbm, z_hbm, p0_hbm, p1_hbm,
               acc, idxr, idxc, rows_v, zrow_v):
    c = lax.axis_index("c")
    s = lax.axis_index("s")
    wid = c * NS + s

    pltpu.sync_copy(z_hbm, zrow_v)

    @pl.loop(0, RPT // C)
    def _zero(i):
        pltpu.sync_copy(zrow_v, acc.at[pl.ds(s * RPT + i * C, C)])

    plsc.subcore_barrier()

    base = wid * EP

    @pl.loop(0, NCH)
    def _chunk(k):
        pltpu.sync_copy(row_hbm.at[pl.ds(base + k * C, C)], idxr)
        pltpu.sync_copy(col_hbm.at[pl.ds(base + k * C, C)], idxc)
        pltpu.sync_copy(hs_hbm.at[idxr], rows_v)
        pltpu.sync_copy(rows_v, acc.at[idxc], add=True)

    plsc.subcore_barrier()

    @pl.when(c == 0)
    def _dump0():
        pltpu.sync_copy(acc.at[pl.ds(s * RPT, RPT)], p0_hbm.at[pl.ds(s * RPT, RPT)])

    @pl.when(c == 1)
    def _dump1():
        pltpu.sync_copy(acc.at[pl.ds(s * RPT, RPT)], p1_hbm.at[pl.ds(s * RPT, RPT)])


_spmm_kernel = pl.kernel(
    _spmm_body,
    out_type=[
        jax.ShapeDtypeStruct((NP, D), jnp.float32),
        jax.ShapeDtypeStruct((NP, D), jnp.float32),
    ],
    mesh=_sc_mesh,
    scratch_types=[
        pltpu.VMEM_SHARED((NP, D), jnp.float32),
        pltpu.VMEM((C,), jnp.int32),
        pltpu.VMEM((C,), jnp.int32),
        pltpu.VMEM((C, D), jnp.float32),
        pltpu.VMEM((C, D), jnp.float32),
    ],
)


def _prep_body(d0_ref, d1_ref, x_ref, dinv_ref, hs0_ref):
    deg = d0_ref[...] + d1_ref[...] + 1.0
    dinv = lax.rsqrt(deg)
    dinv_ref[...] = dinv
    hs0_ref[...] = dinv * x_ref[...]


_prep_kernel = pl.pallas_call(
    _prep_body,
    grid=(NB,),
    in_specs=[
        pl.BlockSpec((RB, 1), lambda i: (i, 0)),
        pl.BlockSpec((RB, 1), lambda i: (i, 0)),
        pl.BlockSpec((RB, D), lambda i: (i, 0)),
    ],
    out_specs=[
        pl.BlockSpec((RB, 1), lambda i: (i, 0)),
        pl.BlockSpec((RB, D), lambda i: (i, 0)),
    ],
    out_shape=[
        jax.ShapeDtypeStruct((NP, 1), jnp.float32),
        jax.ShapeDtypeStruct((NP, D), jnp.float32),
    ],
)


def _mid_body(dinv_ref, p0_ref, p1_ref, hs0_ref, h1_ref, hs1_ref):
    agg = p0_ref[...] + p1_ref[...] + hs0_ref[...]
    dinv = dinv_ref[...]
    h1 = dinv * agg
    h1_ref[...] = h1
    hs1_ref[...] = dinv * h1


_mid_kernel = pl.pallas_call(
    _mid_body,
    grid=(NB,),
    in_specs=[
        pl.BlockSpec((RB, 1), lambda i: (i, 0)),
        pl.BlockSpec((RB, D), lambda i: (i, 0)),
        pl.BlockSpec((RB, D), lambda i: (i, 0)),
        pl.BlockSpec((RB, D), lambda i: (i, 0)),
    ],
    out_specs=[
        pl.BlockSpec((RB, D), lambda i: (i, 0)),
        pl.BlockSpec((RB, D), lambda i: (i, 0)),
    ],
    out_shape=[
        jax.ShapeDtypeStruct((NP, D), jnp.float32),
        jax.ShapeDtypeStruct((NP, D), jnp.float32),
    ],
)


def _out_body(x_ref, h1_ref, q0_ref, q1_ref, hs1_ref, dinv_ref,
              w0_ref, w1_ref, w2_ref, b_ref, o_ref):
    h2 = dinv_ref[...] * (q0_ref[...] + q1_ref[...] + hs1_ref[...])
    acc = jnp.dot(x_ref[...], w0_ref[...], preferred_element_type=jnp.float32)
    acc = acc + jnp.dot(h1_ref[...], w1_ref[...], preferred_element_type=jnp.float32)
    acc = acc + jnp.dot(h2, w2_ref[...], preferred_element_type=jnp.float32)
    o_ref[...] = acc + b_ref[...]


_out_kernel = pl.pallas_call(
    _out_body,
    grid=(NB,),
    in_specs=[
        pl.BlockSpec((RB, D), lambda i: (i, 0)),
        pl.BlockSpec((RB, D), lambda i: (i, 0)),
        pl.BlockSpec((RB, D), lambda i: (i, 0)),
        pl.BlockSpec((RB, D), lambda i: (i, 0)),
        pl.BlockSpec((RB, D), lambda i: (i, 0)),
        pl.BlockSpec((RB, 1), lambda i: (i, 0)),
        pl.BlockSpec((D, D), lambda i: (0, 0)),
        pl.BlockSpec((D, D), lambda i: (0, 0)),
        pl.BlockSpec((D, D), lambda i: (0, 0)),
        pl.BlockSpec((1, D), lambda i: (0, 0)),
    ],
    out_specs=pl.BlockSpec((RB, D), lambda i: (i, 0)),
    out_shape=jax.ShapeDtypeStruct((NP, D), jnp.float32),
)


@jax.jit
def kernel(x, edge_index, W, b):
    row = edge_index[0]
    col = edge_index[1]
    xp = jnp.pad(x, ((0, NP - N), (0, 0)))
    zrows = jnp.zeros((C, D), jnp.float32)

    d0, d1 = _deg_kernel(col)
    dinv, hs0 = _prep_kernel(d0.reshape(NP, 1), d1.reshape(NP, 1), xp)
    p0, p1 = _spmm_kernel(hs0, row, col, zrows)
    h1, hs1 = _mid_kernel(dinv, p0, p1, hs0)
    q0, q1 = _spmm_kernel(hs1, row, col, zrows)
    Wt = W.T
    out = _out_kernel(xp, h1, q0, q1, hs1, dinv,
                      Wt[:D], Wt[D:2 * D], Wt[2 * D:], b.reshape(1, D))
    return out[:N]


# speedup vs baseline: 26.7249x; 2.2457x over previous
import functools

import jax
import jax.numpy as jnp
from jax import lax
from jax.experimental import pallas as pl
from jax.experimental.pallas import tpu as pltpu
from jax.experimental.pallas import tpu_sc as plsc

N = 10000
E = 320000
D = 128
NC = 2
NS = 16
NW = NC * NS
NP = 10240
RPT = NP // NS
EP = E // NW
C = 80
NCH = EP // C
NB = 16
RB = NP // NB

_sc_mesh = plsc.VectorSubcoreMesh(
    core_axis_name="c", subcore_axis_name="s", num_cores=NC, num_subcores=NS
)


def _deg_body(col2_hbm, deg0_hbm, deg1_hbm, acc, idx_a, ones_v, zero_v):
    c = lax.axis_index("c")
    s = lax.axis_index("s")
    wid = c * NS + s

    @pl.loop(0, RPT // 16)
    def _zfill(i):
        zero_v[pl.ds(i * 16, 16)] = jnp.zeros((16,), jnp.float32)

    @pl.loop(0, C // 16)
    def _ofill(i):
        ones_v[pl.ds(i * 16, 16)] = jnp.ones((16,), jnp.float32)

    pltpu.sync_copy(zero_v, acc.at[pl.ds(s * RPT, RPT)])
    pltpu.sync_copy(col2_hbm.at[wid], idx_a)
    plsc.subcore_barrier()

    @pl.loop(0, NCH)
    def _chunk(k):
        pltpu.sync_copy(ones_v, acc.at[idx_a.at[k]], add=True)

    plsc.subcore_barrier()

    @pl.when(c == 0)
    def _dump0():
        pltpu.sync_copy(acc.at[pl.ds(s * RPT, RPT)], deg0_hbm.at[pl.ds(s * RPT, RPT)])

    @pl.when(c == 1)
    def _dump1():
        pltpu.sync_copy(acc.at[pl.ds(s * RPT, RPT)], deg1_hbm.at[pl.ds(s * RPT, RPT)])


_deg_kernel = pl.kernel(
    _deg_body,
    out_type=[
        jax.ShapeDtypeStruct((NP,), jnp.float32),
        jax.ShapeDtypeStruct((NP,), jnp.float32),
    ],
    mesh=_sc_mesh,
    scratch_types=[
        pltpu.VMEM_SHARED((NP,), jnp.float32),
        pltpu.VMEM((NCH, C), jnp.int32),
        pltpu.VMEM((C,), jnp.float32),
        pltpu.VMEM((RPT,), jnp.float32),
    ],
)


def _spmm_body(hs_hbm, eidx_hbm, z_hbm, p0_hbm, p1_hbm,
               acc, i0, i1, i2, i3, rows0, rows1,
               si0, si1, si2, si3, sg0, sg1):
    c = lax.axis_index("c")
    s = lax.axis_index("s")
    wid = c * NS + s

    idx = (i0, i1, i2, i3)
    isems = (si0, si1, si2, si3)
    rows = (rows0, rows1)
    gsems = (sg0, sg1)

    pltpu.sync_copy(z_hbm, acc.at[pl.ds(s * RPT, RPT)])
    plsc.subcore_barrier()

    for j in (0, 1, 2, 3):
        pltpu.async_copy(eidx_hbm.at[wid, j], idx[j], isems[j])
    for b in (0, 1):
        pltpu.make_async_copy(eidx_hbm.at[wid, b], idx[b], isems[b]).wait()
        pltpu.async_copy(hs_hbm.at[idx[b].at[0]], rows[b], gsems[b])

    @pl.loop(0, NCH - 1, step=4)
    def _chunk(k0):
        for u in (0, 1, 2, 3):
            k = k0 + u
            b = u % 2
            j = u % 4
            j2 = (u + 2) % 4
            pltpu.make_async_copy(hs_hbm.at[idx[j].at[0]], rows[b], gsems[b]).wait()
            pltpu.sync_copy(rows[b], acc.at[idx[j].at[1]], add=True)

            @pl.when(k + 4 < NCH)
            def _refill():
                pltpu.async_copy(eidx_hbm.at[wid, k + 4], idx[j], isems[j])

            @pl.when(k + 2 < NCH)
            def _launch():
                pltpu.make_async_copy(eidx_hbm.at[wid, k + 2], idx[j2], isems[j2]).wait()
                pltpu.async_copy(hs_hbm.at[idx[j2].at[0]], rows[b], gsems[b])

    kl = NCH - 1
    bl = kl % 2
    jl = kl % 4
    pltpu.make_async_copy(hs_hbm.at[idx[jl].at[0]], rows[bl], gsems[bl]).wait()
    pltpu.sync_copy(rows[bl], acc.at[idx[jl].at[1]], add=True)

    plsc.subcore_barrier()

    @pl.when(c == 0)
    def _dump0():
        pltpu.sync_copy(acc.at[pl.ds(s * RPT, RPT)], p0_hbm.at[pl.ds(s * RPT, RPT)])

    @pl.when(c == 1)
    def _dump1():
        pltpu.sync_copy(acc.at[pl.ds(s * RPT, RPT)], p1_hbm.at[pl.ds(s * RPT, RPT)])


_spmm_kernel = pl.kernel(
    _spmm_body,
    out_type=[
        jax.ShapeDtypeStruct((NP, D), jnp.float32),
        jax.ShapeDtypeStruct((NP, D), jnp.float32),
    ],
    mesh=_sc_mesh,
    scratch_types=[
        pltpu.VMEM_SHARED((NP, D), jnp.float32),
        pltpu.VMEM((2, C), jnp.int32),
        pltpu.VMEM((2, C), jnp.int32),
        pltpu.VMEM((2, C), jnp.int32),
        pltpu.VMEM((2, C), jnp.int32),
        pltpu.VMEM((C, D), jnp.float32),
        pltpu.VMEM((C, D), jnp.float32),
        pltpu.SemaphoreType.DMA,
        pltpu.SemaphoreType.DMA,
        pltpu.SemaphoreType.DMA,
        pltpu.SemaphoreType.DMA,
        pltpu.SemaphoreType.DMA,
        pltpu.SemaphoreType.DMA,
    ],
)


def _prep_body(d0_ref, d1_ref, x_ref, dinv_ref, hs0_ref):
    deg = d0_ref[...] + d1_ref[...] + 1.0
    dinv = lax.rsqrt(deg)
    dinv_ref[...] = dinv
    hs0_ref[...] = dinv * x_ref[...]


_prep_kernel = pl.pallas_call(
    _prep_body,
    grid=(NB,),
    in_specs=[
        pl.BlockSpec((RB, 1), lambda i: (i, 0)),
        pl.BlockSpec((RB, 1), lambda i: (i, 0)),
        pl.BlockSpec((RB, D), lambda i: (i, 0)),
    ],
    out_specs=[
        pl.BlockSpec((RB, 1), lambda i: (i, 0)),
        pl.BlockSpec((RB, D), lambda i: (i, 0)),
    ],
    out_shape=[
        jax.ShapeDtypeStruct((NP, 1), jnp.float32),
        jax.ShapeDtypeStruct((NP, D), jnp.float32),
    ],
)


def _mid_body(dinv_ref, p0_ref, p1_ref, hs0_ref, h1_ref, hs1_ref):
    agg = p0_ref[...] + p1_ref[...] + hs0_ref[...]
    dinv = dinv_ref[...]
    h1 = dinv * agg
    h1_ref[...] = h1
    hs1_ref[...] = dinv * h1


_mid_kernel = pl.pallas_call(
    _mid_body,
    grid=(NB,),
    in_specs=[
        pl.BlockSpec((RB, 1), lambda i: (i, 0)),
        pl.BlockSpec((RB, D), lambda i: (i, 0)),
        pl.BlockSpec((RB, D), lambda i: (i, 0)),
        pl.BlockSpec((RB, D), lambda i: (i, 0)),
    ],
    out_specs=[
        pl.BlockSpec((RB, D), lambda i: (i, 0)),
        pl.BlockSpec((RB, D), lambda i: (i, 0)),
    ],
    out_shape=[
        jax.ShapeDtypeStruct((NP, D), jnp.float32),
        jax.ShapeDtypeStruct((NP, D), jnp.float32),
    ],
)


def _out_body(x_ref, h1_ref, q0_ref, q1_ref, hs1_ref, dinv_ref,
              w0_ref, w1_ref, w2_ref, b_ref, o_ref):
    h2 = dinv_ref[...] * (q0_ref[...] + q1_ref[...] + hs1_ref[...])
    acc = jnp.dot(x_ref[...], w0_ref[...], preferred_element_type=jnp.float32)
    acc = acc + jnp.dot(h1_ref[...], w1_ref[...], preferred_element_type=jnp.float32)
    acc = acc + jnp.dot(h2, w2_ref[...], preferred_element_type=jnp.float32)
    o_ref[...] = acc + b_ref[...]


_out_kernel = pl.pallas_call(
    _out_body,
    grid=(NB,),
    in_specs=[
        pl.BlockSpec((RB, D), lambda i: (i, 0)),
        pl.BlockSpec((RB, D), lambda i: (i, 0)),
        pl.BlockSpec((RB, D), lambda i: (i, 0)),
        pl.BlockSpec((RB, D), lambda i: (i, 0)),
        pl.BlockSpec((RB, D), lambda i: (i, 0)),
        pl.BlockSpec((RB, 1), lambda i: (i, 0)),
        pl.BlockSpec((D, D), lambda i: (0, 0)),
        pl.BlockSpec((D, D), lambda i: (0, 0)),
        pl.BlockSpec((D, D), lambda i: (0, 0)),
        pl.BlockSpec((1, D), lambda i: (0, 0)),
    ],
    out_specs=pl.BlockSpec((RB, D), lambda i: (i, 0)),
    out_shape=jax.ShapeDtypeStruct((NP, D), jnp.float32),
)


@jax.jit
def kernel(x, edge_index, W, b):
    col2 = edge_index[1].reshape(NW, NCH, C)
    eidx = edge_index.reshape(2, NW, NCH, C).transpose(1, 2, 0, 3)
    xp = jnp.pad(x, ((0, NP - N), (0, 0)))
    zrows = jnp.zeros((RPT, D), jnp.float32)

    d0, d1 = _deg_kernel(col2)
    dinv, hs0 = _prep_kernel(d0.reshape(NP, 1), d1.reshape(NP, 1), xp)
    p0, p1 = _spmm_kernel(hs0, eidx, zrows)
    h1, hs1 = _mid_kernel(dinv, p0, p1, hs0)
    q0, q1 = _spmm_kernel(hs1, eidx, zrows)
    Wt = W.T
    out = _out_kernel(xp, h1, q0, q1, hs1, dinv,
                      Wt[:D], Wt[D:2 * D], Wt[2 * D:], b.reshape(1, D))
    return out[:N]
